# baseline (speedup 1.0000x reference)
import functools
import jax
import jax.numpy as jnp
from jax import lax
from jax.experimental import pallas as pl
from jax.experimental.pallas import tpu as pltpu
from jax.experimental.pallas import tpu_sc as plsc

N = 1024
D_EDGE = 4
BR = 128
NB = N // BR

_info = plsc.get_sparse_core_info()
_NC, _NS, _L = _info.num_cores, _info.num_subcores, _info.num_lanes
_NW = _NC * _NS
_ROWS_PER_W = N // _NW


def _build_sc(adj_hbm, out_hbm, row_buf, out_buf, sem_in, sem_out):
    wid = lax.axis_index("s") * _NC + lax.axis_index("c")
    base_row = wid * _ROWS_PER_W
    lane = lax.iota(jnp.int32, _L)

    def row_body(j, _):
        r = base_row + j
        pltpu.make_async_copy(adj_hbm.at[r], row_buf, sem_in).start()
        pltpu.make_async_copy(adj_hbm.at[r], row_buf, sem_in).wait()

        def grp_body(g, _):
            col = g * _L + lane
            idx0 = col * D_EDGE
            v0 = plsc.load_gather(row_buf, [idx0])
            v1 = plsc.load_gather(row_buf, [idx0 + 1])
            v2 = plsc.load_gather(row_buf, [idx0 + 2])
            v3 = plsc.load_gather(row_buf, [idx0 + 3])
            m = jnp.maximum(jnp.maximum(v0, v1), jnp.maximum(v2, v3))
            w = jnp.where((m != 0.0) & (col != r), 1.0, 0.0)
            out_buf[pl.ds(g * _L, _L)] = w
            return 0

        lax.fori_loop(0, N // _L, grp_body, 0, unroll=False)
        pltpu.make_async_copy(out_buf, out_hbm.at[r], sem_out).start()
        pltpu.make_async_copy(out_buf, out_hbm.at[r], sem_out).wait()
        return 0

    lax.fori_loop(0, _ROWS_PER_W, row_body, 0, unroll=False)


def build_a01(adj_matrix):
    adj_rows = adj_matrix.reshape(N, N * D_EDGE)
    mesh = plsc.VectorSubcoreMesh(core_axis_name="c", subcore_axis_name="s")
    k = functools.partial(
        pl.kernel, mesh=mesh,
        out_type=jax.ShapeDtypeStruct((N, N), jnp.float32),
        scratch_types=[
            pltpu.VMEM((N * D_EDGE,), jnp.float32),
            pltpu.VMEM((N,), jnp.float32),
            pltpu.SemaphoreType.DMA,
            pltpu.SemaphoreType.DMA,
        ],
        compiler_params=pltpu.CompilerParams(needs_layout_passes=False),
    )(_build_sc)
    return k(adj_rows)


def _chebnet_tc(a01f_ref, x_ref, w1_ref, b1_ref, w2_ref, b2_ref,
                out_ref, a01_scr):
    i = pl.program_id(0)

    @pl.when(i < NB)
    def _cast_block():
        a01_scr[pl.ds(i * BR, BR), :] = a01f_ref[...].astype(jnp.bfloat16)

    @pl.when(i == NB)
    def _compute():
        a01 = a01_scr[...]
        ones = jnp.ones((N, 1), jnp.bfloat16)
        deg = jnp.dot(a01, ones, preferred_element_type=jnp.float32)
        dis = jnp.where(deg > 0.0, jax.lax.rsqrt(deg), 0.0)
        x = x_ref[...]

        def smul(v):
            vb = (dis * v).astype(jnp.bfloat16)
            return -dis * jnp.dot(a01, vb, preferred_element_type=jnp.float32)

        def cheb(v, w_ref, b_ref):
            t1 = smul(v)
            t2 = 2.0 * smul(t1) - v
            o = (jnp.dot(v, w_ref[0], preferred_element_type=jnp.float32)
                 + jnp.dot(t1, w_ref[1], preferred_element_type=jnp.float32)
                 + jnp.dot(t2, w_ref[2], preferred_element_type=jnp.float32))
            return o + b_ref[...]

        h = jnp.maximum(cheb(x, w1_ref, b1_ref), 0.0)
        o = cheb(h, w2_ref, b2_ref)
        m = jnp.max(o, axis=1, keepdims=True)
        e = jnp.exp(o - m)
        out_ref[...] = e / jnp.sum(e, axis=1, keepdims=True)


def kernel(feat_matrix, adj_matrix, get_item_index, set_index, val_index,
           mask_matrix, W1, b1, W2, b2):
    n, f0 = feat_matrix.shape
    f1 = W1.shape[-1]
    f2 = W2.shape[-1]
    a01f = build_a01(adj_matrix)
    b1r = b1.reshape(1, f1)
    b2r = b2.reshape(1, f2)

    out = pl.pallas_call(
        _chebnet_tc,
        grid=(NB + 1,),
        in_specs=[
            pl.BlockSpec((BR, n), lambda i: (jnp.minimum(i, NB - 1), 0)),
            pl.BlockSpec((n, f0), lambda i: (0, 0)),
            pl.BlockSpec((W1.shape[0], f0, f1), lambda i: (0, 0, 0)),
            pl.BlockSpec((1, f1), lambda i: (0, 0)),
            pl.BlockSpec((W2.shape[0], f1, f2), lambda i: (0, 0, 0)),
            pl.BlockSpec((1, f2), lambda i: (0, 0)),
        ],
        out_specs=pl.BlockSpec((n, f2), lambda i: (0, 0)),
        out_shape=jax.ShapeDtypeStruct((n, f2), jnp.float32),
        scratch_shapes=[
            pltpu.VMEM((n, n), jnp.bfloat16),
        ],
        compiler_params=pltpu.CompilerParams(
            dimension_semantics=("arbitrary",),
        ),
    )(a01f, feat_matrix, W1, b1r, W2, b2r)
    return out


# speedup vs baseline: 3.9439x; 3.9439x over previous
import jax
import jax.numpy as jnp
from jax.experimental import pallas as pl
from jax.experimental.pallas import tpu as pltpu

N = 1024
D_EDGE = 4
BR = 128
NB = N // BR


def _chebnet_kernel(adj_ref, x_ref, w1_ref, b1_ref, w2_ref, b2_ref,
                    out_ref, a01_scr, deg_scr):
    i = pl.program_id(0)

    @pl.when(i < NB)
    def _build_block():
        a = adj_ref[...]
        m = jnp.maximum(jnp.maximum(a[0], a[1]), jnp.maximum(a[2], a[3]))
        valid = m.astype(jnp.float32) != 0.0
        rows = jax.lax.broadcasted_iota(jnp.int32, (BR, N), 0) + i * BR
        cols = jax.lax.broadcasted_iota(jnp.int32, (BR, N), 1)
        w = jnp.where(valid & (rows != cols), 1.0, 0.0)
        a01_scr[pl.ds(i * BR, BR), :] = w.astype(jnp.bfloat16)
        deg_scr[pl.ds(i * BR, BR), :] = jnp.sum(w, axis=1, keepdims=True)

    @pl.when(i == NB)
    def _compute():
        deg = deg_scr[...]
        dis = jnp.where(deg > 0.0, jax.lax.rsqrt(deg), 0.0)
        a01 = a01_scr[...]
        x = x_ref[...]

        def smul(v):
            vb = (dis * v).astype(jnp.bfloat16)
            return -dis * jnp.dot(a01, vb, preferred_element_type=jnp.float32)

        def cheb(v, w_ref, b_ref):
            t1 = smul(v)
            t2 = 2.0 * smul(t1) - v
            o = (jnp.dot(v, w_ref[0], preferred_element_type=jnp.float32)
                 + jnp.dot(t1, w_ref[1], preferred_element_type=jnp.float32)
                 + jnp.dot(t2, w_ref[2], preferred_element_type=jnp.float32))
            return o + b_ref[...]

        h = jnp.maximum(cheb(x, w1_ref, b1_ref), 0.0)
        o = cheb(h, w2_ref, b2_ref)
        m = jnp.max(o, axis=1, keepdims=True)
        e = jnp.exp(o - m)
        out_ref[...] = e / jnp.sum(e, axis=1, keepdims=True)


def kernel(feat_matrix, adj_matrix, get_item_index, set_index, val_index,
           mask_matrix, W1, b1, W2, b2):
    n, f0 = feat_matrix.shape
    f1 = W1.shape[-1]
    f2 = W2.shape[-1]
    adjt = jnp.transpose(adj_matrix.astype(jnp.bfloat16), (2, 0, 1))
    b1r = b1.reshape(1, f1)
    b2r = b2.reshape(1, f2)

    out = pl.pallas_call(
        _chebnet_kernel,
        grid=(NB + 1,),
        in_specs=[
            pl.BlockSpec((D_EDGE, BR, n), lambda i: (0, jnp.minimum(i, NB - 1), 0)),
            pl.BlockSpec((n, f0), lambda i: (0, 0)),
            pl.BlockSpec((W1.shape[0], f0, f1), lambda i: (0, 0, 0)),
            pl.BlockSpec((1, f1), lambda i: (0, 0)),
            pl.BlockSpec((W2.shape[0], f1, f2), lambda i: (0, 0, 0)),
            pl.BlockSpec((1, f2), lambda i: (0, 0)),
        ],
        out_specs=pl.BlockSpec((n, f2), lambda i: (0, 0)),
        out_shape=jax.ShapeDtypeStruct((n, f2), jnp.float32),
        scratch_shapes=[
            pltpu.VMEM((n, n), jnp.bfloat16),
            pltpu.VMEM((n, 1), jnp.float32),
        ],
        compiler_params=pltpu.CompilerParams(
            dimension_semantics=("arbitrary",),
        ),
    )(adjt, feat_matrix, W1, b1r, W2, b2r)
    return out
